# Initial kernel scaffold; baseline (speedup 1.0000x reference)
import jax
import jax.numpy as jnp
from jax.experimental import pallas as pl


def kernel(token_ids, word_table, special_table):
    raise NotImplementedError("write your pallas kernel here")


# speedup vs baseline: 4.3973x; 4.3973x over previous
import functools

import jax
import jax.numpy as jnp
from jax import lax
from jax.experimental import pallas as pl
from jax.experimental.pallas import tpu as pltpu
from jax.experimental.pallas import tpu_sc as plsc

NUM_SPECIAL_TOKENS = 100
EMBED_DIM = 64

_NC = 2
_NS = 16
_NW = _NC * _NS

_CHUNK = 512
_GSEG = 128
_NSEG = _CHUNK // _GSEG


def _emb_kernel(n_tokens, tok_hbm, word_hbm, spec_hbm, out_hbm,
                tok_v, idx_v, rows_v, spec_v, sem):
    wid = lax.axis_index("s") * _NC + lax.axis_index("c")
    per_w = n_tokens // _NW
    n_chunks = per_w // _CHUNK
    base_w = wid * per_w

    pltpu.sync_copy(spec_hbm, spec_v)

    def chunk_body(g, _):
        base = base_w + g * _CHUNK
        pltpu.sync_copy(tok_hbm.at[pl.ds(base, _CHUNK)], tok_v)

        def idx_body(i, acc):
            def sub_body(k, acc):
                t = tok_v[pl.ds(i * _GSEG + k * 16, 16)]
                spi = jnp.where(t < NUM_SPECIAL_TOKENS, 1, 0)
                idx_v[i, pl.ds(k * 16, 16)] = jnp.where(
                    spi > 0, 0, t + 1 - NUM_SPECIAL_TOKENS)
                return acc | spi
            return lax.fori_loop(0, _GSEG // 16, sub_body, acc)
        acc = lax.fori_loop(0, _NSEG, idx_body,
                            jnp.zeros((16,), jnp.int32))
        chunk_has_sp = acc[0]
        for l in range(1, 16):
            chunk_has_sp = chunk_has_sp | acc[l]

        copies = [
            pltpu.async_copy(word_hbm.at[idx_v.at[j]],
                             rows_v.at[pl.ds(j * _GSEG, _GSEG)], sem)
            for j in range(_NSEG)
        ]
        for c in copies:
            c.wait()

        @pl.when(chunk_has_sp > 0)
        def _():
            def sp_body(i, _):
                t = tok_v[pl.ds(i * 16, 16)]
                spi = jnp.where(t < NUM_SPECIAL_TOKENS, 1, 0)
                g_has = spi[0]
                for l in range(1, 16):
                    g_has = g_has | spi[l]

                @pl.when(g_has > 0)
                def _():
                    idxs = jnp.where(spi > 0, t + 1, 0)
                    row_ids = i * 16 + lax.iota(jnp.int32, 16)

                    def col_body(c, _):
                        cvec = jnp.broadcast_to(c, (16,)).astype(jnp.int32)
                        vals = plsc.load_gather(spec_v, [idxs, cvec])
                        cur = plsc.load_gather(rows_v, [row_ids, cvec])
                        plsc.store_scatter(rows_v, [row_ids, cvec],
                                           cur + vals)
                        return 0
                    lax.fori_loop(0, EMBED_DIM, col_body, 0)
                return 0
            lax.fori_loop(0, _CHUNK // 16, sp_body, 0)

        pltpu.sync_copy(rows_v, out_hbm.at[pl.ds(base, _CHUNK)])
        return 0

    lax.fori_loop(0, n_chunks, chunk_body, 0)


@functools.partial(jax.jit, static_argnames=())
def _emb(tok, word_table, special_table):
    n_tokens = tok.shape[0]
    mesh = plsc.VectorSubcoreMesh(core_axis_name="c", subcore_axis_name="s")
    f = functools.partial(
        pl.kernel,
        mesh=mesh,
        compiler_params=pltpu.CompilerParams(
            needs_layout_passes=False, use_tc_tiling_on_sc=False),
        out_type=jax.ShapeDtypeStruct((n_tokens, EMBED_DIM), jnp.float32),
        scratch_types=[
            pltpu.VMEM((_CHUNK,), jnp.int32),
            pltpu.VMEM((_NSEG, _GSEG), jnp.int32),
            pltpu.VMEM((_CHUNK, EMBED_DIM), jnp.float32),
            pltpu.VMEM(special_table.shape, jnp.float32),
            pltpu.SemaphoreType.DMA,
        ],
    )(functools.partial(_emb_kernel, n_tokens))
    return f(tok, word_table, special_table)


def kernel(token_ids, word_table, special_table):
    tok = token_ids.reshape(-1).astype(jnp.int32)
    out = _emb(tok, word_table.astype(jnp.float32),
               special_table.astype(jnp.float32))
    return out.reshape(*token_ids.shape, EMBED_DIM)


# speedup vs baseline: 4.8247x; 1.0972x over previous
import functools

import jax
import jax.numpy as jnp
from jax import lax
from jax.experimental import pallas as pl
from jax.experimental.pallas import tpu as pltpu
from jax.experimental.pallas import tpu_sc as plsc

NUM_SPECIAL_TOKENS = 100
EMBED_DIM = 64

_NC = 2
_NS = 16
_NW = _NC * _NS

_CHUNK = 256
_GSEG = 128
_NSEG = _CHUNK // _GSEG
_NBUF = 4


def _emb_kernel(n_tokens, tok_hbm, word_hbm, spec_hbm, out_hbm,
                tok_v, idx_v, rows_v, spec_v, flags_v, gsem, tsem, osem):
    wid = lax.axis_index("s") * _NC + lax.axis_index("c")
    per_w = n_tokens // _NW
    n = per_w // _CHUNK
    base_w = wid * per_w

    pltpu.sync_copy(spec_hbm, spec_v)

    def tok_copy(g, b):
        base = base_w + g * _CHUNK
        return pltpu.make_async_copy(
            tok_hbm.at[pl.ds(base, _CHUNK)], tok_v.at[b], tsem.at[b])

    def gather_copy(g, b, j):
        del g
        return pltpu.make_async_copy(
            word_hbm.at[idx_v.at[b, j]],
            rows_v.at[b, pl.ds(j * _GSEG, _GSEG)], gsem.at[b])

    def out_copy(g, b):
        base = base_w + g * _CHUNK
        return pltpu.make_async_copy(
            rows_v.at[b], out_hbm.at[pl.ds(base, _CHUNK)], osem.at[b])

    def compute_indices(b):
        def idx_body(k, acc):
            j = k // (_GSEG // 16)
            kk = k % (_GSEG // 16)
            t = tok_v[b, pl.ds(j * _GSEG + kk * 16, 16)]
            spi = jnp.where(t < NUM_SPECIAL_TOKENS, 1, 0)
            idx_v[b, j, pl.ds(kk * 16, 16)] = jnp.where(
                spi > 0, 0, t + 1 - NUM_SPECIAL_TOKENS)
            return acc | spi
        acc = lax.fori_loop(0, _CHUNK // 16, idx_body,
                            jnp.zeros((16,), jnp.int32))
        flags_v[b, pl.ds(0, 16)] = acc

    def fire_gathers(g, b):
        for j in range(_NSEG):
            gather_copy(g, b, j).start()

    def patch_specials(b):
        acc = flags_v[b, pl.ds(0, 16)]
        chunk_has_sp = acc[0]
        for l in range(1, 16):
            chunk_has_sp = chunk_has_sp | acc[l]

        @pl.when(chunk_has_sp > 0)
        def _():
            def sp_body(i, _):
                t = tok_v[b, pl.ds(i * 16, 16)]
                spi = jnp.where(t < NUM_SPECIAL_TOKENS, 1, 0)
                g_has = spi[0]
                for l in range(1, 16):
                    g_has = g_has | spi[l]

                @pl.when(g_has > 0)
                def _():
                    idxs = jnp.where(spi > 0, t + 1, 0)
                    row_ids = i * 16 + lax.iota(jnp.int32, 16)

                    def col_body(c, _):
                        cvec = jnp.broadcast_to(c, (16,)).astype(jnp.int32)
                        vals = plsc.load_gather(spec_v, [idxs, cvec])
                        cur = plsc.load_gather(rows_v.at[b], [row_ids, cvec])
                        plsc.store_scatter(rows_v.at[b], [row_ids, cvec],
                                           cur + vals)
                        return 0
                    lax.fori_loop(0, EMBED_DIM, col_body, 0)
                return 0
            lax.fori_loop(0, _CHUNK // 16, sp_body, 0)

    for g in range(_NBUF):
        tok_copy(g, g).start()
    for g in range(_NBUF - 1):
        tok_copy(g, g).wait()
        compute_indices(g)
        fire_gathers(g, g)

    def outer(i, _):
        for b in range(_NBUF):
            g = i * _NBUF + b
            for j in range(_NSEG):
                gather_copy(g, b, j).wait()
            patch_specials(b)
            out_copy(g, b).start()

            @pl.when(g + _NBUF < n)
            def _():
                tok_copy(g + _NBUF, b).start()

            b3 = (b + _NBUF - 1) % _NBUF

            @pl.when(g + _NBUF - 1 < n)
            def _():
                @pl.when(g >= 1)
                def _():
                    out_copy(g - 1, b3).wait()
                tok_copy(g + _NBUF - 1, b3).wait()
                compute_indices(b3)
                fire_gathers(g + _NBUF - 1, b3)
        return 0
    lax.fori_loop(0, n // _NBUF, outer, 0)

    for g in range(n - _NBUF, n):
        out_copy(g, g % _NBUF).wait()


@jax.jit
def _emb(tok, word_table, special_table):
    n_tokens = tok.shape[0]
    mesh = plsc.VectorSubcoreMesh(core_axis_name="c", subcore_axis_name="s")
    f = functools.partial(
        pl.kernel,
        mesh=mesh,
        compiler_params=pltpu.CompilerParams(
            needs_layout_passes=False, use_tc_tiling_on_sc=False),
        out_type=jax.ShapeDtypeStruct((n_tokens, EMBED_DIM), jnp.float32),
        scratch_types=[
            pltpu.VMEM((_NBUF, _CHUNK), jnp.int32),
            pltpu.VMEM((_NBUF, _NSEG, _GSEG), jnp.int32),
            pltpu.VMEM((_NBUF, _CHUNK, EMBED_DIM), jnp.float32),
            pltpu.VMEM(special_table.shape, jnp.float32),
            pltpu.VMEM((_NBUF, 16), jnp.int32),
            pltpu.SemaphoreType.DMA((_NBUF,)),
            pltpu.SemaphoreType.DMA((_NBUF,)),
            pltpu.SemaphoreType.DMA((_NBUF,)),
        ],
    )(functools.partial(_emb_kernel, n_tokens))
    return f(tok, word_table, special_table)


def kernel(token_ids, word_table, special_table):
    tok = token_ids.reshape(-1).astype(jnp.int32)
    out = _emb(tok, word_table.astype(jnp.float32),
               special_table.astype(jnp.float32))
    return out.reshape(*token_ids.shape, EMBED_DIM)
